# baseline (speedup 1.0000x reference)
import functools

import jax
import jax.numpy as jnp
from jax import lax
from jax.experimental import pallas as pl
from jax.experimental.pallas import tpu as pltpu
from jax.experimental.pallas import tpu_sc as plsc

_HB = 16


def _sc_gather_body(alpha_hbm, t_hbm, out_hbm, idx_v, rows_v, sem):
    wid = lax.axis_index("s") * 2 + lax.axis_index("c")

    @pl.when(wid == 0)
    def _():
        pltpu.sync_copy(t_hbm, idx_v)
        pltpu.async_copy(alpha_hbm.at[idx_v], rows_v, sem).wait()
        pltpu.sync_copy(rows_v, out_hbm.at[0])


def _tc_body(a_ref, x_ref, n_ref, out_ref, nout_ref):
    a = a_ref[...].reshape(1, 1, 1, 128)
    sa = jnp.sqrt(a)
    sn = jnp.sqrt(1.0 - a)
    n = n_ref[...]
    out_ref[...] = sa * x_ref[...] + sn * n
    nout_ref[...] = n


def kernel(x_0, noise, t, alpha_cumprod):
    B, C, H, W = x_0.shape
    xT = jnp.transpose(x_0, (1, 2, 3, 0))
    nT = jnp.transpose(noise, (1, 2, 3, 0))

    sc_gather = functools.partial(
        pl.kernel,
        mesh=plsc.VectorSubcoreMesh(core_axis_name="c", subcore_axis_name="s"),
        out_type=jax.ShapeDtypeStruct((1, B), jnp.float32),
        scratch_types=[
            pltpu.VMEM((B,), jnp.int32),
            pltpu.VMEM((B,), jnp.float32),
            pltpu.SemaphoreType.DMA,
        ],
    )(_sc_gather_body)
    a_vec = sc_gather(alpha_cumprod, t)

    blk = (C, _HB, W, B)
    bmap = lambda h: (0, h, 0, 0)
    outT, noutT = pl.pallas_call(
        _tc_body,
        grid=(H // _HB,),
        in_specs=[
            pl.BlockSpec((1, B), lambda h: (0, 0)),
            pl.BlockSpec(blk, bmap),
            pl.BlockSpec(blk, bmap),
        ],
        out_specs=[pl.BlockSpec(blk, bmap), pl.BlockSpec(blk, bmap)],
        out_shape=[
            jax.ShapeDtypeStruct((C, H, W, B), x_0.dtype),
            jax.ShapeDtypeStruct((C, H, W, B), x_0.dtype),
        ],
    )(a_vec, xT, nT)
    return (
        jnp.transpose(outT, (3, 0, 1, 2)),
        jnp.transpose(noutT, (3, 0, 1, 2)),
        t,
    )
